# Initial kernel scaffold; baseline (speedup 1.0000x reference)
import jax
import jax.numpy as jnp
from jax.experimental import pallas as pl


def kernel(obs, action, next_obs, reward, termination, W0, b0, W1, b1, W2, b2, W3, b3, W_lat, b_lat, embeddings, is_training):
    raise NotImplementedError("write your pallas kernel here")


# speedup vs baseline: 1.1198x; 1.1198x over previous
import jax
import jax.numpy as jnp
from jax.experimental import pallas as pl
from jax.experimental.pallas import tpu as pltpu

B = 4096
OBS_DIM = 256
ACT_DIM = 64
HID = 2048
LATENT_DIM = 256
NUM_EMBEDDINGS = 1024
COMMITMENT_COST = 0.25

BLK = 512
NBLK = B // BLK


def _layer_kernel(x_ref, w_ref, b_ref, o_ref):
    h = jnp.dot(x_ref[...], w_ref[...], preferred_element_type=jnp.float32)
    o_ref[...] = jnp.maximum(h + b_ref[...], 0.0)


def _dense_relu(x, w, b):
    m, k = x.shape
    n = w.shape[1]
    return pl.pallas_call(
        _layer_kernel,
        grid=(m // BLK,),
        in_specs=[
            pl.BlockSpec((BLK, k), lambda i: (i, 0)),
            pl.BlockSpec((k, n), lambda i: (0, 0)),
            pl.BlockSpec((1, n), lambda i: (0, 0)),
        ],
        out_specs=pl.BlockSpec((BLK, n), lambda i: (i, 0)),
        out_shape=jax.ShapeDtypeStruct((m, n), jnp.float32),
    )(x, w, b.reshape(1, n))


def _vq_kernel(h_ref, wlat_ref, blat_ref, emb_ref, embt_ref, esq_ref,
               q_ref, idx_ref, loss_ref, perp_ref,
               loss_acc, cnt_acc):
    m = pl.program_id(0)

    @pl.when(m == 0)
    def _init():
        loss_acc[0, 0] = 0.0
        cnt_acc[...] = jnp.zeros_like(cnt_acc)

    z = jnp.dot(h_ref[...], wlat_ref[...],
                preferred_element_type=jnp.float32) + blat_ref[...]
    zsq = jnp.sum(z * z, axis=1, keepdims=True)
    d = zsq - 2.0 * jnp.dot(z, emb_ref[...],
                            preferred_element_type=jnp.float32) + esq_ref[...]
    dmin = jnp.min(d, axis=1, keepdims=True)
    lane = jax.lax.broadcasted_iota(jnp.int32, d.shape, 1)
    idx = jnp.min(jnp.where(d == dmin, lane, NUM_EMBEDDINGS), axis=1)
    onehot = (lane == idx[:, None]).astype(jnp.float32)
    q = jnp.dot(onehot, embt_ref[...], preferred_element_type=jnp.float32)
    q_ref[...] = q
    idx_ref[...] = idx[None, :]

    diff = q - z
    loss_acc[0, 0] += jnp.sum(diff * diff)
    cnt_acc[...] += jnp.sum(onehot, axis=0)[None, :]

    @pl.when(m == NBLK - 1)
    def _fini():
        loss_ref[...] = jnp.reshape(
            (COMMITMENT_COST / (B * LATENT_DIM)) * loss_acc[0, 0], (1, 1))
        p = cnt_acc[...] * (1.0 / B)
        ent = jnp.sum(p * jnp.log(p + 1e-10))
        perp_ref[...] = jnp.reshape(jnp.exp(-ent), (1, 1))


def _vq(h, w_lat, b_lat, embeddings):
    esq = jnp.sum(embeddings * embeddings, axis=0, keepdims=True)
    q, idx, loss, perp = pl.pallas_call(
        _vq_kernel,
        grid=(NBLK,),
        in_specs=[
            pl.BlockSpec((BLK, HID), lambda i: (i, 0)),
            pl.BlockSpec((HID, LATENT_DIM), lambda i: (0, 0)),
            pl.BlockSpec((1, LATENT_DIM), lambda i: (0, 0)),
            pl.BlockSpec((LATENT_DIM, NUM_EMBEDDINGS), lambda i: (0, 0)),
            pl.BlockSpec((NUM_EMBEDDINGS, LATENT_DIM), lambda i: (0, 0)),
            pl.BlockSpec((1, NUM_EMBEDDINGS), lambda i: (0, 0)),
        ],
        out_specs=[
            pl.BlockSpec((BLK, LATENT_DIM), lambda i: (i, 0)),
            pl.BlockSpec((1, BLK), lambda i: (0, i)),
            pl.BlockSpec((1, 1), lambda i: (0, 0)),
            pl.BlockSpec((1, 1), lambda i: (0, 0)),
        ],
        out_shape=[
            jax.ShapeDtypeStruct((B, LATENT_DIM), jnp.float32),
            jax.ShapeDtypeStruct((1, B), jnp.int32),
            jax.ShapeDtypeStruct((1, 1), jnp.float32),
            jax.ShapeDtypeStruct((1, 1), jnp.float32),
        ],
        scratch_shapes=[
            pltpu.SMEM((1, 1), jnp.float32),
            pltpu.VMEM((1, NUM_EMBEDDINGS), jnp.float32),
        ],
    )(h, w_lat, b_lat.reshape(1, LATENT_DIM), embeddings,
      embeddings.T, esq)
    return q, idx.reshape(B), loss.reshape(()), perp.reshape(())


def kernel(obs, action, next_obs, reward, termination,
           W0, b0, W1, b1, W2, b2, W3, b3, W_lat, b_lat,
           embeddings, is_training):
    x = jnp.hstack([obs, action, next_obs, reward, termination])
    h = _dense_relu(x, W0, b0)
    h = _dense_relu(h, W1, b1)
    h = _dense_relu(h, W2, b2)
    h = _dense_relu(h, W3, b3)
    q, idx, loss, perp = _vq(h, W_lat, b_lat, embeddings)
    return q, loss, perp, idx
